# baseline (speedup 1.0000x reference)
import functools

import jax
import jax.numpy as jnp
from jax.experimental import pallas as pl
from jax.experimental.pallas import tpu as pltpu

NUM_SEGMENTS = 8


def _pick_block(n, candidates):
    for b in candidates:
        if n % b == 0:
            return b
    return 1


def _pool_mlp_kernel(xb, segb, w1, b1, w2, b2, out, sums, counts, *, block):
    i = pl.program_id(0)
    k = pl.num_programs(0)

    @pl.when(i == 0)
    def _init():
        sums[...] = jnp.zeros_like(sums)
        counts[...] = jnp.zeros_like(counts)

    seg = segb[0, 0, :]
    iota = jax.lax.broadcasted_iota(jnp.int32, (NUM_SEGMENTS, block), 0)
    onehot = (iota == seg[None, :]).astype(jnp.float32)
    xv = xb[...]
    xh = xv.astype(jnp.bfloat16).astype(jnp.float32)
    xm = (xv - xh).astype(jnp.bfloat16).astype(jnp.float32)
    xl = xv - xh - xm
    acc = jax.lax.dot(onehot, xh, preferred_element_type=jnp.float32)
    acc += jax.lax.dot(onehot, xm, preferred_element_type=jnp.float32)
    acc += jax.lax.dot(onehot, xl, preferred_element_type=jnp.float32)
    sums[...] += acc
    counts[...] += jnp.sum(onehot, axis=1, keepdims=True)

    @pl.when(i == k - 1)
    def _finish():
        h = sums[...] / jnp.maximum(counts[...], 1.0)
        h = jax.lax.dot_general(h, w1[...], (((1,), (1,)), ((), ())),
                                precision=jax.lax.Precision.HIGHEST,
                                preferred_element_type=jnp.float32) + b1[...]
        h = jnp.where(h > 0, h, 0.01 * h)
        h = jax.lax.dot_general(h, w2[...], (((1,), (1,)), ((), ())),
                                precision=jax.lax.Precision.HIGHEST,
                                preferred_element_type=jnp.float32) + b2[...]
        h = jnp.where(h > 0, h, 0.01 * h)
        out[...] = h


def _tile_kernel(h, out):
    out[...] = jnp.broadcast_to(h[...][None, :, :], out.shape)


def kernel(x, batch, W1, b1, W2, b2):
    n, c = x.shape
    s = NUM_SEGMENTS

    block = _pick_block(n, (10000, 5000, 2000, 1000, 400, 200, 80, 40, 16, 8))
    k = n // block
    seg3 = batch.reshape(k, 1, block)

    h = pl.pallas_call(
        functools.partial(_pool_mlp_kernel, block=block),
        grid=(k,),
        in_specs=[
            pl.BlockSpec((block, c), lambda i: (i, 0)),
            pl.BlockSpec((1, 1, block), lambda i: (i, 0, 0)),
            pl.BlockSpec((c, c), lambda i: (0, 0)),
            pl.BlockSpec((1, c), lambda i: (0, 0)),
            pl.BlockSpec((c, c), lambda i: (0, 0)),
            pl.BlockSpec((1, c), lambda i: (0, 0)),
        ],
        out_specs=pl.BlockSpec((s, c), lambda i: (0, 0)),
        out_shape=jax.ShapeDtypeStruct((s, c), jnp.float32),
        scratch_shapes=[
            pltpu.VMEM((s, c), jnp.float32),
            pltpu.VMEM((s, 1), jnp.float32),
        ],
    )(x, seg3, W1, b1.reshape(1, c), W2, b2.reshape(1, c))

    r = _pick_block(n, (625, 1250, 2500, 3125, 6250, 5000, 2500, 1250, 1000, 625, 500, 400, 250, 200, 125,
                        100, 80, 50, 40, 25, 20, 16, 10, 8, 5, 4, 2))
    tiles = n // r
    out = pl.pallas_call(
        _tile_kernel,
        grid=(tiles,),
        in_specs=[pl.BlockSpec((s, c), lambda i: (0, 0))],
        out_specs=pl.BlockSpec((r, s, c), lambda i: (i, 0, 0)),
        out_shape=jax.ShapeDtypeStruct((n, s, c), jnp.float32),
    )(h)
    return out.reshape(n * s, c)


# speedup vs baseline: 1.0562x; 1.0562x over previous
import functools

import jax
import jax.numpy as jnp
from jax.experimental import pallas as pl
from jax.experimental.pallas import tpu as pltpu

NUM_SEGMENTS = 8


def _pick_block(n, candidates):
    for b in candidates:
        if n % b == 0:
            return b
    return 1


def _pool_mlp_kernel(xb, segb, w1, b1, w2, b2, out, sums, counts, *, block):
    i = pl.program_id(0)
    k = pl.num_programs(0)

    @pl.when(i == 0)
    def _init():
        sums[...] = jnp.zeros_like(sums)
        counts[...] = jnp.zeros_like(counts)

    seg = segb[0, 0, :]
    iota = jax.lax.broadcasted_iota(jnp.int32, (NUM_SEGMENTS, block), 0)
    onehot = (iota == seg[None, :]).astype(jnp.float32)
    xv = xb[...]
    xh = xv.astype(jnp.bfloat16).astype(jnp.float32)
    xm = (xv - xh).astype(jnp.bfloat16).astype(jnp.float32)
    xl = xv - xh - xm
    acc = jax.lax.dot(onehot, xh, preferred_element_type=jnp.float32)
    acc += jax.lax.dot(onehot, xm, preferred_element_type=jnp.float32)
    acc += jax.lax.dot(onehot, xl, preferred_element_type=jnp.float32)
    sums[...] += acc
    counts[...] += jnp.sum(onehot, axis=1, keepdims=True)

    @pl.when(i == k - 1)
    def _finish():
        h = sums[...] / jnp.maximum(counts[...], 1.0)
        h = jax.lax.dot_general(h, w1[...], (((1,), (1,)), ((), ())),
                                precision=jax.lax.Precision.HIGHEST,
                                preferred_element_type=jnp.float32) + b1[...]
        h = jnp.where(h > 0, h, 0.01 * h)
        h = jax.lax.dot_general(h, w2[...], (((1,), (1,)), ((), ())),
                                precision=jax.lax.Precision.HIGHEST,
                                preferred_element_type=jnp.float32) + b2[...]
        h = jnp.where(h > 0, h, 0.01 * h)
        out[...] = h


def _tile_kernel(h, out):
    out[...] = jnp.broadcast_to(h[...][None, :, :], out.shape)


def kernel(x, batch, W1, b1, W2, b2):
    n, c = x.shape
    s = NUM_SEGMENTS

    block = _pick_block(n, (10000, 5000, 2000, 1000, 400, 200, 80, 40, 16, 8))
    k = n // block
    seg3 = batch.reshape(k, 1, block)

    h = pl.pallas_call(
        functools.partial(_pool_mlp_kernel, block=block),
        grid=(k,),
        in_specs=[
            pl.BlockSpec((block, c), lambda i: (i, 0)),
            pl.BlockSpec((1, 1, block), lambda i: (i, 0, 0)),
            pl.BlockSpec((c, c), lambda i: (0, 0)),
            pl.BlockSpec((1, c), lambda i: (0, 0)),
            pl.BlockSpec((c, c), lambda i: (0, 0)),
            pl.BlockSpec((1, c), lambda i: (0, 0)),
        ],
        out_specs=pl.BlockSpec((s, c), lambda i: (0, 0)),
        out_shape=jax.ShapeDtypeStruct((s, c), jnp.float32),
        scratch_shapes=[
            pltpu.VMEM((s, c), jnp.float32),
            pltpu.VMEM((s, 1), jnp.float32),
        ],
    )(x, seg3, W1, b1.reshape(1, c), W2, b2.reshape(1, c))

    r = _pick_block(n, (2000, 1250, 2500, 3125, 6250, 5000, 2500, 1250, 1000, 625, 500, 400, 250, 200, 125,
                        100, 80, 50, 40, 25, 20, 16, 10, 8, 5, 4, 2))
    tiles = n // r
    out = pl.pallas_call(
        _tile_kernel,
        grid=(tiles,),
        in_specs=[pl.BlockSpec((s, c), lambda i: (0, 0))],
        out_specs=pl.BlockSpec((r, s, c), lambda i: (i, 0, 0)),
        out_shape=jax.ShapeDtypeStruct((n, s, c), jnp.float32),
    )(h)
    return out.reshape(n * s, c)


# speedup vs baseline: 1.0602x; 1.0037x over previous
import functools

import jax
import jax.numpy as jnp
from jax.experimental import pallas as pl
from jax.experimental.pallas import tpu as pltpu

NUM_SEGMENTS = 8


def _pick_block(n, candidates):
    for b in candidates:
        if n % b == 0:
            return b
    return 1


def _pool_mlp_kernel(xb, segb, w1, b1, w2, b2, out, sums, counts, *, block):
    i = pl.program_id(0)
    k = pl.num_programs(0)

    @pl.when(i == 0)
    def _init():
        sums[...] = jnp.zeros_like(sums)
        counts[...] = jnp.zeros_like(counts)

    seg = segb[0, 0, :]
    iota = jax.lax.broadcasted_iota(jnp.int32, (NUM_SEGMENTS, block), 0)
    onehot = (iota == seg[None, :]).astype(jnp.float32)
    xv = xb[...]
    xh = xv.astype(jnp.bfloat16).astype(jnp.float32)
    xm = (xv - xh).astype(jnp.bfloat16).astype(jnp.float32)
    xl = xv - xh - xm
    acc = jax.lax.dot(onehot, xh, preferred_element_type=jnp.float32)
    acc += jax.lax.dot(onehot, xm, preferred_element_type=jnp.float32)
    acc += jax.lax.dot(onehot, xl, preferred_element_type=jnp.float32)
    sums[...] += acc
    counts[...] += jnp.sum(onehot, axis=1, keepdims=True)

    @pl.when(i == k - 1)
    def _finish():
        h = sums[...] / jnp.maximum(counts[...], 1.0)
        h = jax.lax.dot_general(h, w1[...], (((1,), (1,)), ((), ())),
                                precision=jax.lax.Precision.HIGHEST,
                                preferred_element_type=jnp.float32) + b1[...]
        h = jnp.where(h > 0, h, 0.01 * h)
        h = jax.lax.dot_general(h, w2[...], (((1,), (1,)), ((), ())),
                                precision=jax.lax.Precision.HIGHEST,
                                preferred_element_type=jnp.float32) + b2[...]
        h = jnp.where(h > 0, h, 0.01 * h)
        out[...] = h


def _tile_kernel(h, out):
    out[...] = jnp.broadcast_to(h[...][None, :, :], out.shape)


def kernel(x, batch, W1, b1, W2, b2):
    n, c = x.shape
    s = NUM_SEGMENTS

    block = _pick_block(n, (25000, 10000, 5000, 2000, 1000, 400, 200, 80, 40, 16, 8))
    k = n // block
    seg3 = batch.reshape(k, 1, block)

    h = pl.pallas_call(
        functools.partial(_pool_mlp_kernel, block=block),
        grid=(k,),
        in_specs=[
            pl.BlockSpec((block, c), lambda i: (i, 0)),
            pl.BlockSpec((1, 1, block), lambda i: (i, 0, 0)),
            pl.BlockSpec((c, c), lambda i: (0, 0)),
            pl.BlockSpec((1, c), lambda i: (0, 0)),
            pl.BlockSpec((c, c), lambda i: (0, 0)),
            pl.BlockSpec((1, c), lambda i: (0, 0)),
        ],
        out_specs=pl.BlockSpec((s, c), lambda i: (0, 0)),
        out_shape=jax.ShapeDtypeStruct((s, c), jnp.float32),
        scratch_shapes=[
            pltpu.VMEM((s, c), jnp.float32),
            pltpu.VMEM((s, 1), jnp.float32),
        ],
    )(x, seg3, W1, b1.reshape(1, c), W2, b2.reshape(1, c))

    r = _pick_block(n, (1250, 2500, 3125, 6250, 5000, 2500, 1250, 1000, 625, 500, 400, 250, 200, 125,
                        100, 80, 50, 40, 25, 20, 16, 10, 8, 5, 4, 2))
    tiles = n // r
    out = pl.pallas_call(
        _tile_kernel,
        grid=(tiles,),
        in_specs=[pl.BlockSpec((s, c), lambda i: (0, 0))],
        out_specs=pl.BlockSpec((r, s, c), lambda i: (i, 0, 0)),
        out_shape=jax.ShapeDtypeStruct((n, s, c), jnp.float32),
    )(h)
    return out.reshape(n * s, c)
